# Initial kernel scaffold; baseline (speedup 1.0000x reference)
import jax
import jax.numpy as jnp
from jax.experimental import pallas as pl


def kernel(logits):
    raise NotImplementedError("write your pallas kernel here")


# speedup vs baseline: 1.1039x; 1.1039x over previous
import jax
import jax.numpy as jnp
from jax.experimental import pallas as pl

_TEMPERATURE = 0.7
_TOP_P = 0.9
_SAMPLE_KEY = 42
_BIG = 2**30
_ROWS = 8


def _cumsum_last(x):
    n = x.shape[-1]
    k = 1
    while k < n:
        shifted = jnp.concatenate(
            [jnp.zeros(x.shape[:-1] + (k,), x.dtype), x[:, : n - k]], axis=-1
        )
        x = x + shifted
        k *= 2
    return x


def _sample_body(probs_ref, ps_ref, g_ref, out_ref):
    ps = ps_ref[...]
    cs = _cumsum_last(ps)
    kept = jnp.where((cs - ps) > _TOP_P, 0.0, ps)
    s = jnp.sum(kept, axis=-1, keepdims=True)
    q = kept / s
    v = jnp.log(q + 1e-20) + g_ref[...]
    iota = jax.lax.broadcasted_iota(jnp.int32, v.shape, 1)
    maxv = jnp.max(v, axis=-1, keepdims=True)
    jstar = jnp.min(jnp.where(v == maxv, iota, _BIG), axis=-1, keepdims=True)
    vstar = jnp.sum(jnp.where(iota == jstar, ps, 0.0), axis=-1, keepdims=True)
    pr = probs_ref[...]
    cnt_gt = jnp.sum((pr > vstar).astype(jnp.int32), axis=-1, keepdims=True)
    m = jstar - cnt_gt
    eq = pr == vstar
    eqcs = _cumsum_last(eq.astype(jnp.int32))
    tok = jnp.min(jnp.where(eq & (eqcs == m + 1), iota, _BIG), axis=-1)
    out_ref[...] = jnp.broadcast_to(tok[:, None], out_ref.shape).astype(jnp.int32)


def kernel(logits):
    b, v = logits.shape
    probs = jax.nn.softmax(logits / _TEMPERATURE, axis=-1)
    probs_sort = -jax.lax.sort(-probs, dimension=-1)
    g = jax.random.gumbel(jax.random.key(_SAMPLE_KEY), probs.shape, probs.dtype)
    row_spec = pl.BlockSpec((_ROWS, v), lambda i: (i, 0))
    out = pl.pallas_call(
        _sample_body,
        grid=(b // _ROWS,),
        in_specs=[row_spec, row_spec, row_spec],
        out_specs=pl.BlockSpec((_ROWS, 128), lambda i: (i, 0)),
        out_shape=jax.ShapeDtypeStruct((b, 128), jnp.int32),
    )(probs, probs_sort, g)
    return out[:, 0]
